# Initial kernel scaffold; baseline (speedup 1.0000x reference)
import jax
import jax.numpy as jnp
from jax.experimental import pallas as pl


def kernel(x, edge_index, batch, W1, a_src1, a_dst1, b1, W2, a_src2, a_dst2, b2, W3, a_src3, a_dst3, b3, fc1_W, fc1_b, fc2_W, fc2_b):
    raise NotImplementedError("write your pallas kernel here")


# speedup vs baseline: 21.0069x; 21.0069x over previous
import functools

import jax
import jax.numpy as jnp
from jax import lax
from jax.experimental import pallas as pl
from jax.experimental.pallas import tpu as pltpu
from jax.experimental.pallas import tpu_sc as plsc

N_PAD = 50176
E_PAD = 802816
CHUNK = 6272
CHUNK_T = CHUNK + 16
NB = 1792
EB = 2048
NW = 32


def _make_gather():
    D = 128
    per_w = E_PAD // NW
    n_blk = per_w // 128
    mesh = plsc.VectorSubcoreMesh(core_axis_name="c", subcore_axis_name="s")

    @functools.partial(
        pl.kernel, mesh=mesh,
        out_type=jax.ShapeDtypeStruct((E_PAD, D), jnp.float32),
        scratch_types=[
            pltpu.VMEM((n_blk, 128), jnp.int32),
            pltpu.VMEM((128, D), jnp.float32),
            pltpu.VMEM((128, D), jnp.float32),
            pltpu.SemaphoreType.DMA,
            pltpu.SemaphoreType.DMA,
        ],
    )
    def gather_k(table_hbm, idx_hbm, out_hbm, idx_v, row0, row1, sem0, sem1):
        wid = lax.axis_index("s") * 2 + lax.axis_index("c")
        pltpu.sync_copy(idx_hbm.at[wid], idx_v)
        rows = (row0, row1)
        sems = (sem0, sem1)
        pltpu.async_copy(table_hbm.at[idx_v.at[0]], row0, sem0)

        def body(g, _):
            for b in (0, 1):
                j = g * 2 + b

                @pl.when(j + 1 < n_blk)
                def _fire():
                    pltpu.async_copy(table_hbm.at[idx_v.at[j + 1]],
                                     rows[1 - b], sems[1 - b])

                pltpu.make_async_copy(table_hbm.at[idx_v.at[j]],
                                      rows[b], sems[b]).wait()
                pltpu.sync_copy(
                    rows[b],
                    out_hbm.at[pl.ds(wid * per_w + j * 128, 128), :])
            return _

        lax.fori_loop(0, n_blk // 2, body, None)

    return gather_k


def _make_scatter(D):
    per_t = E_PAD // 16
    n_blk = per_t // 128
    rows_o = CHUNK // 16
    mesh = plsc.VectorSubcoreMesh(core_axis_name="c", subcore_axis_name="s")

    @functools.partial(
        pl.kernel, mesh=mesh,
        compiler_params=pltpu.CompilerParams(use_tc_tiling_on_sc=False),
        out_type=jax.ShapeDtypeStruct((8 * CHUNK, D), jnp.float32),
        scratch_types=[
            pltpu.VMEM((1, 128), jnp.int32),
            pltpu.VMEM((1, 128), jnp.int32),
            pltpu.VMEM((128,), jnp.int32),
            pltpu.VMEM((128, D), jnp.float32),
            pltpu.VMEM((128, D), jnp.float32),
            pltpu.VMEM_SHARED((CHUNK_T, D), jnp.float32),
            pltpu.SemaphoreType.DMA,
            pltpu.SemaphoreType.DMA,
            pltpu.SemaphoreType.DMA,
            pltpu.SemaphoreType.DMA,
        ],
    )
    def scatter_k(msg_hbm, dix_hbm, zeros_hbm, out_hbm,
                  i0, i1, adj_v, m0, m1, acc_sh, s0, s1, t0, t1):
        cid = lax.axis_index("c")
        sid = lax.axis_index("s")
        msgs = (m0, m1)
        sems = (s0, s1)
        ibufs = (i0, i1)
        isems = (t0, t1)
        for p in (0, 1, 2, 3):
            base = (cid * 4 + p) * CHUNK
            pltpu.sync_copy(zeros_hbm.at[pl.ds(sid * rows_o, rows_o), :],
                            acc_sh.at[pl.ds(sid * rows_o, rows_o), :])

            @pl.when(sid == 0)
            def _trash():
                pltpu.sync_copy(zeros_hbm.at[pl.ds(CHUNK, 16), :],
                                acc_sh.at[pl.ds(CHUNK, 16), :])

            plsc.subcore_barrier()
            pltpu.async_copy(msg_hbm.at[pl.ds(sid * per_t, 128), :], m0, s0)
            pltpu.async_copy(dix_hbm.at[sid * n_blk], i0, t0)

            def body(g, _):
                for b in (0, 1):
                    j = g * 2 + b

                    @pl.when(j + 1 < n_blk)
                    def _fire():
                        pltpu.async_copy(
                            msg_hbm.at[pl.ds(sid * per_t + (j + 1) * 128,
                                             128), :],
                            msgs[1 - b], sems[1 - b])
                        pltpu.async_copy(dix_hbm.at[sid * n_blk + j + 1],
                                         ibufs[1 - b], isems[1 - b])

                    pltpu.make_async_copy(
                        msg_hbm.at[pl.ds(sid * per_t + j * 128, 128), :],
                        msgs[b], sems[b]).wait()
                    pltpu.make_async_copy(dix_hbm.at[sid * n_blk + j],
                                          ibufs[b], isems[b]).wait()
                    for t in range(8):
                        v = ibufs[b][0, pl.ds(t * 16, 16)]
                        inr = (v >= base) & (v < base + CHUNK)
                        adj_v[pl.ds(t * 16, 16)] = jnp.where(
                            inr, v - base, CHUNK)
                    pltpu.sync_copy(msgs[b], acc_sh.at[adj_v], add=True)
                return _

            lax.fori_loop(0, n_blk // 2, body, None)
            plsc.subcore_barrier()
            pltpu.sync_copy(
                acc_sh.at[pl.ds(sid * rows_o, rows_o), :],
                out_hbm.at[pl.ds(base + sid * rows_o, rows_o), :])
            plsc.subcore_barrier()

    return scatter_k


def _pre_body(x_ref, W_ref, T_ref):
    h = x_ref[...] @ W_ref[...]
    T_ref[...] = jnp.concatenate(
        [h, jnp.zeros((h.shape[0], 128 - h.shape[1]), jnp.float32)], axis=1)


def _mid_body(num_ref, den_ref, R_ref, b_ref, W_ref, T_ref):
    den = den_ref[...] @ R_ref[...]
    g = num_ref[...] / (den + 1e-16) + b_ref[0:1, :]
    hin = jnp.where(g > 0, g, jnp.exp(g) - 1.0)
    T_ref[...] = hin @ W_ref[...]


def _msg_body(hs_ref, hd_ref, As_ref, Ad_ref, R_ref, out_ref):
    hc = R_ref.shape[1]
    a = hs_ref[...] @ As_ref[...] + hd_ref[...] @ Ad_ref[...]
    e = jnp.where(a > 0, a, 0.2 * a)
    w = jnp.exp(e)
    mh = (w @ R_ref[...]) * hs_ref[:, 0:hc]
    z8 = jnp.zeros((a.shape[0], 8), jnp.float32)
    out_ref[...] = jnp.concatenate([w, mh, z8], axis=1)


def _pool_body(num_ref, den_ref, R_ref, b_ref, batch_ref, pool_ref):
    i = pl.program_id(0)
    den = den_ref[...] @ R_ref[...]
    g = num_ref[...] / (den + 1e-16) + b_ref[0:1, :]
    ones = jnp.ones((g.shape[0], 1), jnp.float32)
    z7 = jnp.zeros((g.shape[0], 7), jnp.float32)
    gcat = jnp.concatenate([g, ones, z7], axis=1)
    bv = batch_ref[0, 0, :]
    gid = lax.broadcasted_iota(jnp.int32, (64, g.shape[0]), 0)
    oh = (bv[None, :] == gid).astype(jnp.float32)

    @pl.when(i == 0)
    def _init():
        pool_ref[...] = jnp.zeros_like(pool_ref)

    pool_ref[...] += oh @ gcat


def _mlp_body(pool_ref, W1_ref, b1_ref, W2_ref, b2_ref, out_ref):
    counts = pool_ref[:, 128:129]
    pooled = pool_ref[:, 0:128] / jnp.maximum(counts, 1.0)
    z = jnp.maximum(pooled @ W1_ref[...] + b1_ref[0:1, :], 0.0)
    logits = z @ W2_ref[...] + b2_ref[0:1, :]
    l = logits - jnp.max(logits, axis=1, keepdims=True)
    out_ref[...] = l - jnp.log(jnp.sum(jnp.exp(l), axis=1, keepdims=True))


def _full(shape):
    return pl.BlockSpec(shape, lambda i: tuple(0 for _ in shape))


def _rows(w):
    return pl.BlockSpec((NB, w), lambda i: (i, 0))


def _bd(att, rows):
    hc = att.shape[0] * att.shape[1]
    c = att.shape[1]
    i = jnp.arange(hc)
    out = jnp.zeros((rows, 8), jnp.float32)
    return out.at[i, i // c].set(att.reshape(-1))


def _expander(c):
    hc = 8 * c
    i = jnp.arange(hc)
    return jnp.zeros((8, hc), jnp.float32).at[i // c, i].set(1.0)


def kernel(x, edge_index, batch, W1, a_src1, a_dst1, b1, W2, a_src2, a_dst2,
           b2, W3, a_src3, a_dst3, b3, fc1_W, fc1_b, fc2_W, fc2_b):
    N = x.shape[0]
    E = edge_index.shape[1]
    G = 64

    xp = jnp.pad(x, ((0, N_PAD - N), (0, 0)))
    src = edge_index[0].astype(jnp.int32)
    dst = edge_index[1].astype(jnp.int32)
    src_g = jnp.pad(src, (0, E_PAD - E)).reshape(NW, E_PAD // NW // 128, 128)
    dst_g = jnp.pad(dst, (0, E_PAD - E)).reshape(NW, E_PAD // NW // 128, 128)
    dst_s = jnp.pad(dst, (0, E_PAD - E),
                    constant_values=N_PAD).reshape(E_PAD // 128, 1, 128)
    batch3 = jnp.pad(batch.astype(jnp.int32), (0, N_PAD - N),
                     constant_values=G).reshape(N_PAD // NB, 1, NB)
    z80 = jnp.zeros((CHUNK_T, 80), jnp.float32)
    z144 = jnp.zeros((CHUNK_T, 144), jnp.float32)
    R8 = _expander(8)
    R16 = _expander(16)
    b1t = jnp.tile(b1[None, :], (8, 1))
    b2t = jnp.tile(b2[None, :], (8, 1))
    b3t = jnp.tile(b3[None, :], (8, 1))

    gather = _make_gather()
    scatter80 = _make_scatter(80)
    scatter144 = _make_scatter(144)

    def msg_call(hs, hd, As, Ad, R, d):
        return pl.pallas_call(
            _msg_body,
            grid=(E_PAD // EB,),
            in_specs=[pl.BlockSpec((EB, 128), lambda i: (i, 0)),
                      pl.BlockSpec((EB, 128), lambda i: (i, 0)),
                      _full((128, 8)), _full((128, 8)), _full(R.shape)],
            out_specs=pl.BlockSpec((EB, d), lambda i: (i, 0)),
            out_shape=jax.ShapeDtypeStruct((E_PAD, d), jnp.float32),
        )(hs, hd, As, Ad, R)

    def tc_layer(body, ins, widths):
        return pl.pallas_call(
            body,
            grid=(N_PAD // NB,),
            in_specs=[_rows(w) for w in widths]
            + [_full(a.shape) for a in ins[len(widths):]],
            out_specs=pl.BlockSpec((NB, 128), lambda i: (i, 0)),
            out_shape=jax.ShapeDtypeStruct((N_PAD, 128), jnp.float32),
        )(*ins)

    T1 = tc_layer(_pre_body, [xp, W1], [2])
    hs1 = gather(T1, src_g)
    hd1 = gather(T1, dst_g)
    msg1 = msg_call(hs1, hd1, _bd(a_src1, 128), _bd(a_dst1, 128), R8, 80)
    acc1 = scatter80(msg1, dst_s, z80)

    T2 = tc_layer(_mid_body,
                  [acc1[:, 8:72], acc1[:, 0:8], R8, b1t, W2], [64, 8])
    hs2 = gather(T2, src_g)
    hd2 = gather(T2, dst_g)
    msg2 = msg_call(hs2, hd2, _bd(a_src2, 128), _bd(a_dst2, 128), R16, 144)
    acc2 = scatter144(msg2, dst_s, z144)

    T3 = tc_layer(_mid_body,
                  [acc2[:, 8:136], acc2[:, 0:8], R16, b2t, W3], [128, 8])
    hs3 = gather(T3, src_g)
    hd3 = gather(T3, dst_g)
    msg3 = msg_call(hs3, hd3, _bd(a_src3, 128), _bd(a_dst3, 128), R16, 144)
    acc3 = scatter144(msg3, dst_s, z144)

    pool = pl.pallas_call(
        _pool_body,
        grid=(N_PAD // NB,),
        in_specs=[pl.BlockSpec((NB, 128), lambda i: (i, 0)),
                  pl.BlockSpec((NB, 8), lambda i: (i, 0)),
                  _full(R16.shape), _full(b3t.shape),
                  pl.BlockSpec((1, 1, NB), lambda i: (i, 0, 0))],
        out_specs=pl.BlockSpec((64, 136), lambda i: (0, 0)),
        out_shape=jax.ShapeDtypeStruct((64, 136), jnp.float32),
    )(acc3[:, 8:136], acc3[:, 0:8], R16, b3t, batch3)

    out = pl.pallas_call(
        _mlp_body,
        grid=(1,),
        in_specs=[_full((64, 136)), _full(fc1_W.shape), _full((8, 32)),
                  _full(fc2_W.shape), _full((8, 10))],
        out_specs=_full((64, 10)),
        out_shape=jax.ShapeDtypeStruct((64, 10), jnp.float32),
    )(pool, fc1_W, jnp.tile(fc1_b[None, :], (8, 1)),
      fc2_W, jnp.tile(fc2_b[None, :], (8, 1)))
    return out


# speedup vs baseline: 21.0254x; 1.0009x over previous
import functools

import jax
import jax.numpy as jnp
from jax import lax
from jax.experimental import pallas as pl
from jax.experimental.pallas import tpu as pltpu
from jax.experimental.pallas import tpu_sc as plsc

N_PAD = 50176
E_PAD = 802816
CHUNK = 6272
CHUNK_T = CHUNK + 16
NB = 1792
EB = 2048
NW = 32


def _make_gather():
    D = 128
    per_w = E_PAD // NW
    n_blk = per_w // 128
    mesh = plsc.VectorSubcoreMesh(core_axis_name="c", subcore_axis_name="s")

    @functools.partial(
        pl.kernel, mesh=mesh,
        out_type=jax.ShapeDtypeStruct((E_PAD, D), jnp.float32),
        scratch_types=[
            pltpu.VMEM((n_blk, 128), jnp.int32),
            pltpu.VMEM((128, D), jnp.float32),
            pltpu.VMEM((128, D), jnp.float32),
            pltpu.VMEM((128, D), jnp.float32),
            pltpu.VMEM((128, D), jnp.float32),
        ] + [pltpu.SemaphoreType.DMA] * 8,
    )
    def gather_k(table_hbm, idx_hbm, out_hbm, idx_v,
                 row0, row1, row2, row3, g0, g1, g2, g3, o0, o1, o2, o3):
        wid = lax.axis_index("s") * 2 + lax.axis_index("c")
        pltpu.sync_copy(idx_hbm.at[wid], idx_v)
        rows = (row0, row1, row2, row3)
        gsem = (g0, g1, g2, g3)
        osem = (o0, o1, o2, o3)
        for b in (0, 1, 2):
            pltpu.async_copy(table_hbm.at[idx_v.at[b]], rows[b], gsem[b])

        def out_dst(j):
            return out_hbm.at[pl.ds(wid * per_w + j * 128, 128), :]

        def body(g, _):
            for b in (0, 1, 2, 3):
                j = g * 4 + b
                pltpu.make_async_copy(table_hbm.at[idx_v.at[j]],
                                      rows[b], gsem[b]).wait()
                pltpu.async_copy(rows[b], out_dst(j), osem[b])
                b3 = (b + 3) % 4

                @pl.when(j + 3 < n_blk)
                def _fire():
                    @pl.when(j >= 1)
                    def _drain():
                        pltpu.make_async_copy(rows[b3], out_dst(j - 1),
                                              osem[b3]).wait()

                    pltpu.async_copy(table_hbm.at[idx_v.at[j + 3]],
                                     rows[b3], gsem[b3])
            return _

        lax.fori_loop(0, n_blk // 4, body, None)
        for k in (4, 3, 2, 1):
            j = n_blk - k
            pltpu.make_async_copy(rows[j % 4], out_dst(j),
                                  osem[j % 4]).wait()

    return gather_k


def _make_scatter(D):
    per_t = E_PAD // 16
    n_blk = per_t // 128
    rows_o = CHUNK // 16
    mesh = plsc.VectorSubcoreMesh(core_axis_name="c", subcore_axis_name="s")

    @functools.partial(
        pl.kernel, mesh=mesh,
        compiler_params=pltpu.CompilerParams(use_tc_tiling_on_sc=False),
        out_type=jax.ShapeDtypeStruct((8 * CHUNK, D), jnp.float32),
        scratch_types=[
            pltpu.VMEM((1, 128), jnp.int32),
            pltpu.VMEM((1, 128), jnp.int32),
            pltpu.VMEM((128,), jnp.int32),
            pltpu.VMEM((128,), jnp.int32),
            pltpu.VMEM((128, D), jnp.float32),
            pltpu.VMEM((128, D), jnp.float32),
            pltpu.VMEM_SHARED((CHUNK_T, D), jnp.float32),
        ] + [pltpu.SemaphoreType.DMA] * 6,
    )
    def scatter_k(msg_hbm, dix_hbm, zeros_hbm, out_hbm,
                  i0, i1, a0, a1, m0, m1, acc_sh, s0, s1, t0, t1, v0, v1):
        cid = lax.axis_index("c")
        sid = lax.axis_index("s")
        msgs = (m0, m1)
        sems = (s0, s1)
        ibufs = (i0, i1)
        isems = (t0, t1)
        abufs = (a0, a1)
        vsems = (v0, v1)
        for p in (0, 1, 2, 3):
            base = (cid * 4 + p) * CHUNK
            pltpu.sync_copy(zeros_hbm.at[pl.ds(sid * rows_o, rows_o), :],
                            acc_sh.at[pl.ds(sid * rows_o, rows_o), :])

            @pl.when(sid == 0)
            def _trash():
                pltpu.sync_copy(zeros_hbm.at[pl.ds(CHUNK, 16), :],
                                acc_sh.at[pl.ds(CHUNK, 16), :])

            plsc.subcore_barrier()
            pltpu.async_copy(msg_hbm.at[pl.ds(sid * per_t, 128), :], m0, s0)
            pltpu.async_copy(dix_hbm.at[sid * n_blk], i0, t0)

            def body(g, _):
                for b in (0, 1):
                    j = g * 2 + b

                    @pl.when(j + 1 < n_blk)
                    def _fire():
                        @pl.when(j >= 1)
                        def _free():
                            pltpu.make_async_copy(
                                msgs[1 - b], acc_sh.at[abufs[1 - b]],
                                vsems[1 - b]).wait()

                        pltpu.async_copy(
                            msg_hbm.at[pl.ds(sid * per_t + (j + 1) * 128,
                                             128), :],
                            msgs[1 - b], sems[1 - b])
                        pltpu.async_copy(dix_hbm.at[sid * n_blk + j + 1],
                                         ibufs[1 - b], isems[1 - b])

                    pltpu.make_async_copy(
                        msg_hbm.at[pl.ds(sid * per_t + j * 128, 128), :],
                        msgs[b], sems[b]).wait()
                    pltpu.make_async_copy(dix_hbm.at[sid * n_blk + j],
                                          ibufs[b], isems[b]).wait()
                    for t in range(8):
                        v = ibufs[b][0, pl.ds(t * 16, 16)]
                        inr = (v >= base) & (v < base + CHUNK)
                        abufs[b][pl.ds(t * 16, 16)] = jnp.where(
                            inr, v - base, CHUNK)
                    pltpu.async_copy(msgs[b], acc_sh.at[abufs[b]],
                                     vsems[b], add=True)
                return _

            lax.fori_loop(0, n_blk // 2, body, None)
            for b in (0, 1):
                pltpu.make_async_copy(msgs[b], acc_sh.at[abufs[b]],
                                      vsems[b]).wait()
            plsc.subcore_barrier()
            pltpu.sync_copy(
                acc_sh.at[pl.ds(sid * rows_o, rows_o), :],
                out_hbm.at[pl.ds(base + sid * rows_o, rows_o), :])
            plsc.subcore_barrier()

    return scatter_k


def _pre_body(x_ref, W_ref, T_ref):
    h = x_ref[...] @ W_ref[...]
    T_ref[...] = jnp.concatenate(
        [h, jnp.zeros((h.shape[0], 128 - h.shape[1]), jnp.float32)], axis=1)


def _mid_body(num_ref, den_ref, R_ref, b_ref, W_ref, T_ref):
    den = den_ref[...] @ R_ref[...]
    g = num_ref[...] / (den + 1e-16) + b_ref[0:1, :]
    hin = jnp.where(g > 0, g, jnp.exp(g) - 1.0)
    T_ref[...] = hin @ W_ref[...]


def _msg_body(hs_ref, hd_ref, As_ref, Ad_ref, R_ref, out_ref):
    hc = R_ref.shape[1]
    a = hs_ref[...] @ As_ref[...] + hd_ref[...] @ Ad_ref[...]
    e = jnp.where(a > 0, a, 0.2 * a)
    w = jnp.exp(e)
    mh = (w @ R_ref[...]) * hs_ref[:, 0:hc]
    z8 = jnp.zeros((a.shape[0], 8), jnp.float32)
    out_ref[...] = jnp.concatenate([w, mh, z8], axis=1)


def _pool_body(num_ref, den_ref, R_ref, b_ref, batch_ref, pool_ref):
    i = pl.program_id(0)
    den = den_ref[...] @ R_ref[...]
    g = num_ref[...] / (den + 1e-16) + b_ref[0:1, :]
    ones = jnp.ones((g.shape[0], 1), jnp.float32)
    z7 = jnp.zeros((g.shape[0], 7), jnp.float32)
    gcat = jnp.concatenate([g, ones, z7], axis=1)
    bv = batch_ref[0, 0, :]
    gid = lax.broadcasted_iota(jnp.int32, (64, g.shape[0]), 0)
    oh = (bv[None, :] == gid).astype(jnp.float32)

    @pl.when(i == 0)
    def _init():
        pool_ref[...] = jnp.zeros_like(pool_ref)

    pool_ref[...] += oh @ gcat


def _mlp_body(pool_ref, W1_ref, b1_ref, W2_ref, b2_ref, out_ref):
    counts = pool_ref[:, 128:129]
    pooled = pool_ref[:, 0:128] / jnp.maximum(counts, 1.0)
    z = jnp.maximum(pooled @ W1_ref[...] + b1_ref[0:1, :], 0.0)
    logits = z @ W2_ref[...] + b2_ref[0:1, :]
    l = logits - jnp.max(logits, axis=1, keepdims=True)
    out_ref[...] = l - jnp.log(jnp.sum(jnp.exp(l), axis=1, keepdims=True))


def _full(shape):
    return pl.BlockSpec(shape, lambda i: tuple(0 for _ in shape))


def _rows(w):
    return pl.BlockSpec((NB, w), lambda i: (i, 0))


def _bd(att, rows):
    hc = att.shape[0] * att.shape[1]
    c = att.shape[1]
    i = jnp.arange(hc)
    out = jnp.zeros((rows, 8), jnp.float32)
    return out.at[i, i // c].set(att.reshape(-1))


def _expander(c):
    hc = 8 * c
    i = jnp.arange(hc)
    return jnp.zeros((8, hc), jnp.float32).at[i // c, i].set(1.0)


def kernel(x, edge_index, batch, W1, a_src1, a_dst1, b1, W2, a_src2, a_dst2,
           b2, W3, a_src3, a_dst3, b3, fc1_W, fc1_b, fc2_W, fc2_b):
    N = x.shape[0]
    E = edge_index.shape[1]
    G = 64

    xp = jnp.pad(x, ((0, N_PAD - N), (0, 0)))
    src = edge_index[0].astype(jnp.int32)
    dst = edge_index[1].astype(jnp.int32)
    src_g = jnp.pad(src, (0, E_PAD - E)).reshape(NW, E_PAD // NW // 128, 128)
    dst_g = jnp.pad(dst, (0, E_PAD - E)).reshape(NW, E_PAD // NW // 128, 128)
    dst_s = jnp.pad(dst, (0, E_PAD - E),
                    constant_values=N_PAD).reshape(E_PAD // 128, 1, 128)
    batch3 = jnp.pad(batch.astype(jnp.int32), (0, N_PAD - N),
                     constant_values=G).reshape(N_PAD // NB, 1, NB)
    z80 = jnp.zeros((CHUNK_T, 80), jnp.float32)
    z144 = jnp.zeros((CHUNK_T, 144), jnp.float32)
    R8 = _expander(8)
    R16 = _expander(16)
    b1t = jnp.tile(b1[None, :], (8, 1))
    b2t = jnp.tile(b2[None, :], (8, 1))
    b3t = jnp.tile(b3[None, :], (8, 1))

    gather = _make_gather()
    scatter80 = _make_scatter(80)
    scatter144 = _make_scatter(144)

    def msg_call(hs, hd, As, Ad, R, d):
        return pl.pallas_call(
            _msg_body,
            grid=(E_PAD // EB,),
            in_specs=[pl.BlockSpec((EB, 128), lambda i: (i, 0)),
                      pl.BlockSpec((EB, 128), lambda i: (i, 0)),
                      _full((128, 8)), _full((128, 8)), _full(R.shape)],
            out_specs=pl.BlockSpec((EB, d), lambda i: (i, 0)),
            out_shape=jax.ShapeDtypeStruct((E_PAD, d), jnp.float32),
        )(hs, hd, As, Ad, R)

    def tc_layer(body, ins, widths):
        return pl.pallas_call(
            body,
            grid=(N_PAD // NB,),
            in_specs=[_rows(w) for w in widths]
            + [_full(a.shape) for a in ins[len(widths):]],
            out_specs=pl.BlockSpec((NB, 128), lambda i: (i, 0)),
            out_shape=jax.ShapeDtypeStruct((N_PAD, 128), jnp.float32),
        )(*ins)

    T1 = tc_layer(_pre_body, [xp, W1], [2])
    hs1 = gather(T1, src_g)
    hd1 = gather(T1, dst_g)
    msg1 = msg_call(hs1, hd1, _bd(a_src1, 128), _bd(a_dst1, 128), R8, 80)
    acc1 = scatter80(msg1, dst_s, z80)

    T2 = tc_layer(_mid_body,
                  [acc1[:, 8:72], acc1[:, 0:8], R8, b1t, W2], [64, 8])
    hs2 = gather(T2, src_g)
    hd2 = gather(T2, dst_g)
    msg2 = msg_call(hs2, hd2, _bd(a_src2, 128), _bd(a_dst2, 128), R16, 144)
    acc2 = scatter144(msg2, dst_s, z144)

    T3 = tc_layer(_mid_body,
                  [acc2[:, 8:136], acc2[:, 0:8], R16, b2t, W3], [128, 8])
    hs3 = gather(T3, src_g)
    hd3 = gather(T3, dst_g)
    msg3 = msg_call(hs3, hd3, _bd(a_src3, 128), _bd(a_dst3, 128), R16, 144)
    acc3 = scatter144(msg3, dst_s, z144)

    pool = pl.pallas_call(
        _pool_body,
        grid=(N_PAD // NB,),
        in_specs=[pl.BlockSpec((NB, 128), lambda i: (i, 0)),
                  pl.BlockSpec((NB, 8), lambda i: (i, 0)),
                  _full(R16.shape), _full(b3t.shape),
                  pl.BlockSpec((1, 1, NB), lambda i: (i, 0, 0))],
        out_specs=pl.BlockSpec((64, 136), lambda i: (0, 0)),
        out_shape=jax.ShapeDtypeStruct((64, 136), jnp.float32),
    )(acc3[:, 8:136], acc3[:, 0:8], R16, b3t, batch3)

    out = pl.pallas_call(
        _mlp_body,
        grid=(1,),
        in_specs=[_full((64, 136)), _full(fc1_W.shape), _full((8, 32)),
                  _full(fc2_W.shape), _full((8, 10))],
        out_specs=_full((64, 10)),
        out_shape=jax.ShapeDtypeStruct((64, 10), jnp.float32),
    )(pool, fc1_W, jnp.tile(fc1_b[None, :], (8, 1)),
      fc2_W, jnp.tile(fc2_b[None, :], (8, 1)))
    return out
